# Initial kernel scaffold; baseline (speedup 1.0000x reference)
import jax
import jax.numpy as jnp
from jax.experimental import pallas as pl


def kernel(cosine, label):
    raise NotImplementedError("write your pallas kernel here")


# speedup vs baseline: 2.9378x; 2.9378x over previous
import functools

import jax
import jax.numpy as jnp
from jax import lax
from jax.experimental import pallas as pl
from jax.experimental.pallas import tpu as pltpu
from jax.experimental.pallas import tpu_sc as plsc

S = 64.0
M = 0.4
A = 1.3
K = 0.1
B = 1024
C = 100000

_CHUNK = 128


def _sc_gather_chunks(cosine_chunks, label):
    info = plsc.get_sparse_core_info()
    nc, ns = info.num_cores, info.num_subcores
    nw = nc * ns
    bpw = B // nw

    mesh = plsc.VectorSubcoreMesh(core_axis_name="c", subcore_axis_name="s")

    @functools.partial(
        pl.kernel,
        out_type=jax.ShapeDtypeStruct((B, _CHUNK), jnp.float32),
        mesh=mesh,
        scratch_types=[
            pltpu.VMEM((bpw,), jnp.int32),
            pltpu.VMEM((bpw,), jnp.int32),
            pltpu.VMEM((bpw, _CHUNK), jnp.float32),
            pltpu.SemaphoreType.DMA,
        ],
    )
    def k(chunks_hbm, label_hbm, tgt_hbm, lab_v, idx_v, rows_v, sem):
        wid = lax.axis_index("s") * nc + lax.axis_index("c")
        base = wid * bpw
        pltpu.sync_copy(label_hbm.at[pl.ds(base, bpw)], lab_v)
        for j in range(bpw // 16):
            lab = lab_v[pl.ds(j * 16, 16)]
            safe = jnp.where(lab == -1, 0, lab)
            rows = base + j * 16 + lax.iota(jnp.int32, 16)
            flat = rows * C + safe
            idx_v[pl.ds(j * 16, 16)] = lax.shift_right_logical(flat, 7)
        pltpu.async_copy(chunks_hbm.at[idx_v], rows_v, sem).wait()
        pltpu.sync_copy(rows_v, tgt_hbm.at[pl.ds(base, bpw)])

    return k(cosine_chunks, label)


_BR = 256
_BC = 2048


def _arccos(t):
    ax = jnp.abs(t)
    p = jnp.float32(-0.0012624911)
    for coef in (0.0066700901, -0.0170881256, 0.0308918810, -0.0501743046,
                 0.0889789874, -0.2145988016, 1.5707963050):
        p = p * ax + jnp.float32(coef)
    r = jnp.sqrt(jnp.maximum(1.0 - ax, 0.0)) * p
    return jnp.where(t >= 0, r, jnp.float32(3.14159265358979) - r)


def _tc_body(lab_ref, chunk_ref, x_ref, o_ref):
    j = pl.program_id(1)
    x = x_ref[...]
    c = jnp.clip(x, -1.0, 1.0)
    lab = lab_ref[...]
    safe = jnp.where(lab == -1, 0, lab)
    i = pl.program_id(0)
    rows = i * _BR + lax.broadcasted_iota(jnp.int32, (_BR, 1), 0)
    off = jnp.bitwise_and(rows * C + safe, _CHUNK - 1)
    lane = lax.broadcasted_iota(jnp.int32, (_BR, _CHUNK), 1)
    t = jnp.sum(jnp.where(lane == off, chunk_ref[...], 0.0), axis=1,
                keepdims=True)
    theta = _arccos(jnp.clip(t, -1.0, 1.0))
    adj = jnp.where(lab != -1, (M + K * A) - K * theta, 0.0) * S
    col = j * _BC + lax.broadcasted_iota(jnp.int32, (_BR, _BC), 1)
    o_ref[...] = c * S - jnp.where(col == lab, adj, 0.0)


def kernel(cosine, label):
    label = label.astype(jnp.int32)
    chunks = jnp.reshape(cosine, (B * C // _CHUNK, _CHUNK))
    tchunk = _sc_gather_chunks(chunks, label)

    grid = (B // _BR, pl.cdiv(C, _BC))
    out = pl.pallas_call(
        _tc_body,
        grid=grid,
        in_specs=[
            pl.BlockSpec((_BR, 1), lambda i, j: (i, 0)),
            pl.BlockSpec((_BR, _CHUNK), lambda i, j: (i, 0)),
            pl.BlockSpec((_BR, _BC), lambda i, j: (i, j)),
        ],
        out_specs=pl.BlockSpec((_BR, _BC), lambda i, j: (i, j)),
        out_shape=jax.ShapeDtypeStruct((B, C), jnp.float32),
    )(label[:, None], tchunk, cosine)
    return out


# speedup vs baseline: 4.5945x; 1.5639x over previous
import functools

import jax
import jax.numpy as jnp
from jax import lax
from jax.experimental import pallas as pl
from jax.experimental.pallas import tpu as pltpu
from jax.experimental.pallas import tpu_sc as plsc

S = 64.0
M = 0.4
A = 1.3
K = 0.1
B = 1024
C = 100000

_CHUNK = 128


def _sc_gather_chunks(cosine_chunks, label):
    info = plsc.get_sparse_core_info()
    nc, ns = info.num_cores, info.num_subcores
    nw = nc * ns
    bpw = B // nw

    mesh = plsc.VectorSubcoreMesh(core_axis_name="c", subcore_axis_name="s")

    @functools.partial(
        pl.kernel,
        out_type=jax.ShapeDtypeStruct((B, _CHUNK), jnp.float32),
        mesh=mesh,
        scratch_types=[
            pltpu.VMEM((bpw,), jnp.int32),
            pltpu.VMEM((bpw,), jnp.int32),
            pltpu.VMEM((bpw, _CHUNK), jnp.float32),
            pltpu.SemaphoreType.DMA,
        ],
    )
    def k(chunks_hbm, label_hbm, tgt_hbm, lab_v, idx_v, rows_v, sem):
        wid = lax.axis_index("s") * nc + lax.axis_index("c")
        base = wid * bpw
        pltpu.sync_copy(label_hbm.at[pl.ds(base, bpw)], lab_v)
        for j in range(bpw // 16):
            lab = lab_v[pl.ds(j * 16, 16)]
            safe = jnp.where(lab == -1, 0, lab)
            rows = base + j * 16 + lax.iota(jnp.int32, 16)
            flat = rows * C + safe
            idx_v[pl.ds(j * 16, 16)] = lax.shift_right_logical(flat, 7)
        pltpu.async_copy(chunks_hbm.at[idx_v], rows_v, sem).wait()
        pltpu.sync_copy(rows_v, tgt_hbm.at[pl.ds(base, bpw)])

    return k(cosine_chunks, label)


_BR = 256
_BC = 2048


def _arccos(t):
    ax = jnp.abs(t)
    p = jnp.float32(-0.0012624911)
    for coef in (0.0066700901, -0.0170881256, 0.0308918810, -0.0501743046,
                 0.0889789874, -0.2145988016, 1.5707963050):
        p = p * ax + jnp.float32(coef)
    r = jnp.sqrt(jnp.maximum(1.0 - ax, 0.0)) * p
    return jnp.where(t >= 0, r, jnp.float32(3.14159265358979) - r)


def _tc_body(lab_ref, chunk_ref, x_ref, o_ref):
    j = pl.program_id(1)
    x = x_ref[...]
    c = jnp.clip(x, -1.0, 1.0)
    lab = lab_ref[...]
    safe = jnp.where(lab == -1, 0, lab)
    i = pl.program_id(0)
    rows = i * _BR + lax.broadcasted_iota(jnp.int32, (_BR, 1), 0)
    off = jnp.bitwise_and(rows * C + safe, _CHUNK - 1)
    lane = lax.broadcasted_iota(jnp.int32, (_BR, _CHUNK), 1)
    t = jnp.sum(jnp.where(lane == off, chunk_ref[...], 0.0), axis=1,
                keepdims=True)
    theta = _arccos(jnp.clip(t, -1.0, 1.0))
    adj = jnp.where(lab != -1, (M + K * A) - K * theta, 0.0) * S
    col = j * _BC + lax.broadcasted_iota(jnp.int32, (_BR, _BC), 1)
    o_ref[...] = c * S - jnp.where(col == lab, adj, 0.0)


def _tc_body_b(lab_ref, x_ref, o_ref):
    j = pl.program_id(1)
    x = x_ref[...]
    lab = lab_ref[...]
    col = j * _BC + lax.broadcasted_iota(jnp.int32, (_BR, _BC), 1)
    mask = col == lab
    t = jnp.sum(jnp.where(mask, x, 0.0), axis=1, keepdims=True)
    theta = _arccos(jnp.clip(t, -1.0, 1.0))
    adj = jnp.where(lab != -1, (M + K * A) - K * theta, 0.0) * S
    c = jnp.clip(x, -1.0, 1.0)
    o_ref[...] = c * S - jnp.where(mask, adj, 0.0)


def kernel(cosine, label):
    label = label.astype(jnp.int32)
    grid = (B // _BR, pl.cdiv(C, _BC))
    out = pl.pallas_call(
        _tc_body_b,
        grid=grid,
        in_specs=[
            pl.BlockSpec((_BR, 1), lambda i, j: (i, 0)),
            pl.BlockSpec((_BR, _BC), lambda i, j: (i, j)),
        ],
        out_specs=pl.BlockSpec((_BR, _BC), lambda i, j: (i, j)),
        out_shape=jax.ShapeDtypeStruct((B, C), jnp.float32),
    )(label[:, None], cosine)
    return out
